# baseline (speedup 1.0000x reference)
import functools

import jax
import jax.numpy as jnp
from jax import lax
from jax.experimental import pallas as pl
from jax.experimental.pallas import tpu as pltpu
from jax.experimental.pallas import tpu_sc as plsc

VOCAB = 1000000
EMBED_DIM = 128
BATCH = 4096
HIST_LEN = 200

NC = 2
NS = 16
NW = NC * NS

B = BATCH * HIST_LEN
ROWS_PER_W = B // NW
CHUNK = 128
NCH = ROWS_PER_W // CHUNK
NBUF = 4

_mesh = plsc.VectorSubcoreMesh(core_axis_name="c", subcore_axis_name="s")


@functools.partial(
    pl.kernel,
    out_type=jax.ShapeDtypeStruct((B, EMBED_DIM), jnp.float32),
    mesh=_mesh,
    scratch_types=[
        pltpu.VMEM((NCH, CHUNK), jnp.int32),
        pltpu.VMEM((CHUNK, EMBED_DIM), jnp.float32),
        pltpu.VMEM((CHUNK, EMBED_DIM), jnp.float32),
        pltpu.VMEM((CHUNK, EMBED_DIM), jnp.float32),
        pltpu.VMEM((CHUNK, EMBED_DIM), jnp.float32),
        pltpu.SemaphoreType.DMA,
        pltpu.SemaphoreType.DMA,
        pltpu.SemaphoreType.DMA,
        pltpu.SemaphoreType.DMA,
        pltpu.SemaphoreType.DMA,
        pltpu.SemaphoreType.DMA,
        pltpu.SemaphoreType.DMA,
        pltpu.SemaphoreType.DMA,
    ],
)
def _gather_kernel(table_hbm, idx_hbm, out_hbm, idx_v, r0, r1, r2, r3,
                   g0, g1, g2, g3, o0, o1, o2, o3):
    rows = [r0, r1, r2, r3]
    gs = [g0, g1, g2, g3]
    os_ = [o0, o1, o2, o3]

    wid = lax.axis_index("s") * NC + lax.axis_index("c")
    base = wid * ROWS_PER_W

    def start_gather(c, b):
        pltpu.async_copy(table_hbm.at[idx_v.at[c]], rows[b], gs[b])

    def wait_gather(c, b):
        pltpu.make_async_copy(table_hbm.at[idx_v.at[c]], rows[b], gs[b]).wait()

    def start_out(c, b):
        pltpu.async_copy(rows[b], out_hbm.at[pl.ds(base + c * CHUNK, CHUNK)],
                         os_[b])

    def wait_out(c, b):
        pltpu.make_async_copy(
            rows[b], out_hbm.at[pl.ds(base + c * CHUNK, CHUNK)], os_[b]).wait()

    pltpu.sync_copy(idx_hbm.at[wid], idx_v)

    start_gather(0, 0)
    start_gather(1, 1)
    start_gather(2, 2)
    wait_gather(0, 0)
    start_out(0, 0)
    start_gather(3, 3)
    wait_gather(1, 1)
    start_out(1, 1)
    wait_out(0, 0)
    start_gather(4, 0)
    wait_gather(2, 2)
    start_out(2, 2)
    wait_out(1, 1)
    start_gather(5, 1)
    wait_gather(3, 3)
    start_out(3, 3)

    @pl.loop(4, NCH, step=4)
    def _(g):
        for j in range(4):
            s = g + j
            bn = (j + 2) % 4

            @pl.when(s + 2 < NCH)
            def _():
                wait_out(s - 2, bn)
                start_gather(s + 2, bn)

            wait_gather(s, j)
            start_out(s, j)

    for j in range(4):
        wait_out(NCH - 4 + j, j)


def kernel(token_seq, table):
    idx = token_seq.reshape(NW, NCH, CHUNK)
    out = _gather_kernel(table, idx)
    return out.reshape(BATCH, HIST_LEN, EMBED_DIM)
